# baseline (speedup 1.0000x reference)
import functools

import jax
import jax.numpy as jnp
from jax import lax
from jax.experimental import pallas as pl
from jax.experimental.pallas import tpu as pltpu
from jax.experimental.pallas import tpu_sc as plsc

N = 10000
E = 160000
F = 256
FH = 128

N_ACC = 10112
ROW_BLK = 128
N_BLKS = N_ACC // ROW_BLK
TILES = 16
ROWS_PER_TILE = N_ACC // TILES

E_PAD = 163840
SLAB = 128
N_SLABS = E_PAD // SLAB
SLABS_PER_TILE = N_SLABS // TILES
IDXC = 16

GS = 64
GSPT = E_PAD // (TILES * GS)
GIDXC = 40

_mesh = plsc.VectorSubcoreMesh(core_axis_name="c", subcore_axis_name="s")


@functools.partial(
    pl.kernel,
    out_type=jax.ShapeDtypeStruct((2, N_ACC, FH), jnp.float32),
    mesh=_mesh,
    scratch_types=[
        pltpu.VMEM((SLABS_PER_TILE * SLAB,), jnp.int32),
        pltpu.VMEM((SLAB, FH), jnp.float32),
        pltpu.VMEM_SHARED((N_ACC, FH), jnp.float32),
    ],
)
def _deg_kernel(edges_hbm, ones_hbm, zeros_hbm, out_hbm, idx_v, ones_v, acc):
    c = lax.axis_index("c")
    s = lax.axis_index("s")
    rbase = s * ROWS_PER_TILE
    ebase = s * SLABS_PER_TILE * SLAB
    pltpu.sync_copy(zeros_hbm, acc.at[pl.ds(rbase, ROWS_PER_TILE)])
    pltpu.sync_copy(ones_hbm, ones_v)

    @pl.when(c == 0)
    def _():
        pltpu.sync_copy(edges_hbm.at[0].at[pl.ds(ebase, SLABS_PER_TILE * SLAB)],
                        idx_v)

    @pl.when(c == 1)
    def _():
        pltpu.sync_copy(edges_hbm.at[1].at[pl.ds(ebase, SLABS_PER_TILE * SLAB)],
                        idx_v)

    plsc.subcore_barrier()

    def body(j, _):
        pltpu.sync_copy(ones_v, acc.at[idx_v.at[pl.ds(j * SLAB, SLAB)]], add=True)
        return ()

    lax.fori_loop(0, SLABS_PER_TILE, body, ())
    plsc.subcore_barrier()

    @pl.when(c == 0)
    def _():
        pltpu.sync_copy(acc.at[pl.ds(rbase, ROWS_PER_TILE)],
                        out_hbm.at[0].at[pl.ds(rbase, ROWS_PER_TILE)])

    @pl.when(c == 1)
    def _():
        pltpu.sync_copy(acc.at[pl.ds(rbase, ROWS_PER_TILE)],
                        out_hbm.at[1].at[pl.ds(rbase, ROWS_PER_TILE)])


@functools.partial(
    pl.kernel,
    out_type=jax.ShapeDtypeStruct((2, N_ACC, FH), jnp.float32),
    mesh=_mesh,
    scratch_types=[
        pltpu.VMEM((GIDXC * GS,), jnp.int32),
        pltpu.VMEM((GIDXC * GS,), jnp.int32),
        pltpu.VMEM((GS, FH), jnp.float32),
        pltpu.VMEM((GS, FH), jnp.float32),
        pltpu.VMEM_SHARED((N_ACC, FH), jnp.float32),
        pltpu.SemaphoreType.DMA,
        pltpu.SemaphoreType.DMA,
    ],
)
def _agg_kernel(y_hbm, edges_hbm, zeros_hbm, out_hbm,
                src_c, dst_c, b0, b1, acc, gsem, ssem):
    bufs = (b0, b1)
    c = lax.axis_index("c")
    s = lax.axis_index("s")
    rbase = s * ROWS_PER_TILE
    pltpu.sync_copy(zeros_hbm, acc.at[pl.ds(rbase, ROWS_PER_TILE)])
    plsc.subcore_barrier()

    def _gather(j, buf):
        pltpu.async_copy(
            y_hbm.at[c].at[src_c.at[pl.ds(j * GS, GS)]], buf, gsem)

    def _gwait(j, buf):
        pltpu.make_async_copy(
            y_hbm.at[c].at[src_c.at[pl.ds(j * GS, GS)]], buf, gsem).wait()

    def _sfire(j, buf):
        pltpu.async_copy(buf, acc.at[dst_c.at[pl.ds(j * GS, GS)]], ssem,
                         add=True)

    def _swait(j, buf):
        pltpu.make_async_copy(buf, acc.at[dst_c.at[pl.ds(j * GS, GS)]],
                              ssem).wait()

    NBUF = len(bufs)

    def chunk_body(k, _):
        ebase = (s * GSPT + k * GIDXC) * GS
        pltpu.sync_copy(edges_hbm.at[0].at[pl.ds(ebase, GIDXC * GS)], src_c)
        pltpu.sync_copy(edges_hbm.at[1].at[pl.ds(ebase, GIDXC * GS)], dst_c)
        for b in range(NBUF):
            _gather(b, bufs[b])

        def body(jj, _):
            for b in range(NBUF):
                j = NBUF * jj + b
                _gwait(j, bufs[b])
                _sfire(j, bufs[b])
            for b in range(NBUF):
                j = NBUF * jj + b
                _swait(j, bufs[b])

                @pl.when(jj < GIDXC // NBUF - 1)
                def _():
                    _gather(j + NBUF, bufs[b])

            return ()

        lax.fori_loop(0, GIDXC // NBUF, body, ())
        return ()

    lax.fori_loop(0, GSPT // GIDXC, chunk_body, ())
    plsc.subcore_barrier()
    pltpu.sync_copy(acc.at[pl.ds(rbase, ROWS_PER_TILE)],
                    out_hbm.at[c].at[pl.ds(rbase, ROWS_PER_TILE)])


def _norm(deg_blk):
    return lax.rsqrt(jnp.clip(deg_blk[:, :1], 1.0, None))


def _mm1_body(h_ref, deg_ref, w_ref, out_ref):
    x = h_ref[...] * _norm(deg_ref[0])
    y = jnp.dot(x, w_ref[...], preferred_element_type=jnp.float32)
    out_ref[0] = y[:, :FH]
    out_ref[1] = y[:, FH:]


def _mm1(h_pad, deg, W):
    return pl.pallas_call(
        _mm1_body,
        grid=(N_BLKS,),
        in_specs=[
            pl.BlockSpec((ROW_BLK, F), lambda i: (i, 0)),
            pl.BlockSpec((1, ROW_BLK, FH), lambda i: (0, i, 0)),
            pl.BlockSpec((F, F), lambda i: (0, 0)),
        ],
        out_specs=pl.BlockSpec((2, ROW_BLK, FH), lambda i: (0, i, 0)),
        out_shape=jax.ShapeDtypeStruct((2, N_ACC, FH), jnp.float32),
    )(h_pad, deg, W)


def _mm2_body(a_ref, deg_ref, b_ref, w_ref, out_ref):
    i = pl.program_id(0)
    x = jnp.concatenate([a_ref[0], a_ref[1]], axis=1)
    norm_dst = _norm(deg_ref[1])
    norm_src = _norm(deg_ref[0])
    x1 = jax.nn.relu(x * norm_dst + b_ref[...])
    rows = i * ROW_BLK + lax.broadcasted_iota(jnp.int32, (ROW_BLK, 1), 0)
    x1 = jnp.where(rows < N, x1, 0.0)
    y = jnp.dot(x1 * norm_src, w_ref[...], preferred_element_type=jnp.float32)
    out_ref[0] = y[:, :FH]
    out_ref[1] = y[:, FH:]


def _mm2(a1, deg, b1, W):
    return pl.pallas_call(
        _mm2_body,
        grid=(N_BLKS,),
        in_specs=[
            pl.BlockSpec((2, ROW_BLK, FH), lambda i: (0, i, 0)),
            pl.BlockSpec((2, ROW_BLK, FH), lambda i: (0, i, 0)),
            pl.BlockSpec((1, F), lambda i: (0, 0)),
            pl.BlockSpec((F, F), lambda i: (0, 0)),
        ],
        out_specs=pl.BlockSpec((2, ROW_BLK, FH), lambda i: (0, i, 0)),
        out_shape=jax.ShapeDtypeStruct((2, N_ACC, FH), jnp.float32),
    )(a1, deg, b1, W)


def _final_body(a_ref, deg_ref, b_ref, out_ref):
    x = jnp.concatenate([a_ref[0], a_ref[1]], axis=1)
    out_ref[...] = x * _norm(deg_ref[1]) + b_ref[...]


def _final(a2, deg, b2):
    return pl.pallas_call(
        _final_body,
        grid=(N_BLKS,),
        in_specs=[
            pl.BlockSpec((2, ROW_BLK, FH), lambda i: (0, i, 0)),
            pl.BlockSpec((2, ROW_BLK, FH), lambda i: (0, i, 0)),
            pl.BlockSpec((1, F), lambda i: (0, 0)),
        ],
        out_specs=pl.BlockSpec((ROW_BLK, F), lambda i: (i, 0)),
        out_shape=jax.ShapeDtypeStruct((N, F), jnp.float32),
    )(a2, deg, b2)


def kernel(h, edge_index, W1, b1, W2, b2):
    src = edge_index[0]
    dst = edge_index[1]
    pad = jnp.full((E_PAD - E,), N, dtype=jnp.int32)
    src_r = jnp.concatenate([src, pad])
    dst_r = jnp.concatenate([dst, pad])
    edges = jnp.stack([src_r, dst_r])

    ones_vals = jnp.ones((SLAB, FH), jnp.float32)
    zeros_slab = jnp.zeros((ROWS_PER_TILE, FH), jnp.float32)
    h_pad = jnp.concatenate([h, jnp.zeros((N_ACC - N, F), h.dtype)], axis=0)

    deg = _deg_kernel(edges, ones_vals, zeros_slab)
    y1 = _mm1(h_pad, deg, W1)
    a1 = _agg_kernel(y1, edges, zeros_slab)
    y2 = _mm2(a1, deg, b1.reshape(1, F), W2)
    a2 = _agg_kernel(y2, edges, zeros_slab)
    return _final(a2, deg, b2.reshape(1, F))


# speedup vs baseline: 1.0279x; 1.0279x over previous
import functools

import jax
import jax.numpy as jnp
from jax import lax
from jax.experimental import pallas as pl
from jax.experimental.pallas import tpu as pltpu
from jax.experimental.pallas import tpu_sc as plsc

N = 10000
E = 160000
F = 256
FH = 128

N_ACC = 10112
ROW_BLK = 128
N_BLKS = N_ACC // ROW_BLK
TILES = 16
ROWS_PER_TILE = N_ACC // TILES

E_PAD = 163840
SLAB = 128
N_SLABS = E_PAD // SLAB
SLABS_PER_TILE = N_SLABS // TILES
IDXC = 16

GS = 64
GSPT = E_PAD // (TILES * GS)
GIDXC = 40

_mesh = plsc.VectorSubcoreMesh(core_axis_name="c", subcore_axis_name="s")


@functools.partial(
    pl.kernel,
    out_type=jax.ShapeDtypeStruct((2, N_ACC, FH), jnp.float32),
    mesh=_mesh,
    scratch_types=[
        pltpu.VMEM((SLABS_PER_TILE * SLAB,), jnp.int32),
        pltpu.VMEM((SLAB, FH), jnp.float32),
        pltpu.VMEM_SHARED((N_ACC, FH), jnp.float32),
    ],
)
def _deg_kernel(edges_hbm, ones_hbm, zeros_hbm, out_hbm, idx_v, ones_v, acc):
    c = lax.axis_index("c")
    s = lax.axis_index("s")
    rbase = s * ROWS_PER_TILE
    ebase = s * SLABS_PER_TILE * SLAB
    pltpu.sync_copy(zeros_hbm, acc.at[pl.ds(rbase, ROWS_PER_TILE)])
    pltpu.sync_copy(ones_hbm, ones_v)

    @pl.when(c == 0)
    def _():
        pltpu.sync_copy(edges_hbm.at[0].at[pl.ds(ebase, SLABS_PER_TILE * SLAB)],
                        idx_v)

    @pl.when(c == 1)
    def _():
        pltpu.sync_copy(edges_hbm.at[1].at[pl.ds(ebase, SLABS_PER_TILE * SLAB)],
                        idx_v)

    plsc.subcore_barrier()

    def body(j, _):
        pltpu.sync_copy(ones_v, acc.at[idx_v.at[pl.ds(j * SLAB, SLAB)]], add=True)
        return ()

    lax.fori_loop(0, SLABS_PER_TILE, body, ())
    plsc.subcore_barrier()

    @pl.when(c == 0)
    def _():
        pltpu.sync_copy(acc.at[pl.ds(rbase, ROWS_PER_TILE)],
                        out_hbm.at[0].at[pl.ds(rbase, ROWS_PER_TILE)])

    @pl.when(c == 1)
    def _():
        pltpu.sync_copy(acc.at[pl.ds(rbase, ROWS_PER_TILE)],
                        out_hbm.at[1].at[pl.ds(rbase, ROWS_PER_TILE)])


@functools.partial(
    pl.kernel,
    out_type=jax.ShapeDtypeStruct((2, N_ACC, FH), jnp.float32),
    mesh=_mesh,
    scratch_types=[
        pltpu.VMEM((GIDXC * GS,), jnp.int32),
        pltpu.VMEM((GIDXC * GS,), jnp.int32),
        pltpu.VMEM((GS, FH), jnp.float32),
        pltpu.VMEM((GS, FH), jnp.float32),
        pltpu.VMEM((GS, FH), jnp.float32),
        pltpu.VMEM((GS, FH), jnp.float32),
        pltpu.VMEM_SHARED((N_ACC, FH), jnp.float32),
        pltpu.SemaphoreType.DMA,
        pltpu.SemaphoreType.DMA,
    ],
)
def _agg_kernel(y_hbm, edges_hbm, zeros_hbm, out_hbm,
                src_c, dst_c, b0, b1, b2, b3, acc, gsem, ssem):
    bufs = (b0, b1, b2, b3)
    c = lax.axis_index("c")
    s = lax.axis_index("s")
    rbase = s * ROWS_PER_TILE
    pltpu.sync_copy(zeros_hbm, acc.at[pl.ds(rbase, ROWS_PER_TILE)])
    plsc.subcore_barrier()

    def _gather(j, buf):
        pltpu.async_copy(
            y_hbm.at[c].at[src_c.at[pl.ds(j * GS, GS)]], buf, gsem)

    def _gwait(j, buf):
        pltpu.make_async_copy(
            y_hbm.at[c].at[src_c.at[pl.ds(j * GS, GS)]], buf, gsem).wait()

    def _sfire(j, buf):
        pltpu.async_copy(buf, acc.at[dst_c.at[pl.ds(j * GS, GS)]], ssem,
                         add=True)

    def _swait(j, buf):
        pltpu.make_async_copy(buf, acc.at[dst_c.at[pl.ds(j * GS, GS)]],
                              ssem).wait()

    NBUF = len(bufs)

    def chunk_body(k, _):
        ebase = (s * GSPT + k * GIDXC) * GS
        pltpu.sync_copy(edges_hbm.at[0].at[pl.ds(ebase, GIDXC * GS)], src_c)
        pltpu.sync_copy(edges_hbm.at[1].at[pl.ds(ebase, GIDXC * GS)], dst_c)
        for b in range(NBUF):
            _gather(b, bufs[b])

        def body(jj, _):
            for b in range(NBUF):
                j = NBUF * jj + b
                _gwait(j, bufs[b])
                _sfire(j, bufs[b])
            for b in range(NBUF):
                j = NBUF * jj + b
                _swait(j, bufs[b])

                @pl.when(jj < GIDXC // NBUF - 1)
                def _():
                    _gather(j + NBUF, bufs[b])

            return ()

        lax.fori_loop(0, GIDXC // NBUF, body, ())
        return ()

    lax.fori_loop(0, GSPT // GIDXC, chunk_body, ())
    plsc.subcore_barrier()
    pltpu.sync_copy(acc.at[pl.ds(rbase, ROWS_PER_TILE)],
                    out_hbm.at[c].at[pl.ds(rbase, ROWS_PER_TILE)])


def _norm(deg_blk):
    return lax.rsqrt(jnp.clip(deg_blk[:, :1], 1.0, None))


def _mm1_body(h_ref, deg_ref, w_ref, out_ref):
    x = h_ref[...] * _norm(deg_ref[0])
    y = jnp.dot(x, w_ref[...], preferred_element_type=jnp.float32)
    out_ref[0] = y[:, :FH]
    out_ref[1] = y[:, FH:]


def _mm1(h_pad, deg, W):
    return pl.pallas_call(
        _mm1_body,
        grid=(N_BLKS,),
        in_specs=[
            pl.BlockSpec((ROW_BLK, F), lambda i: (i, 0)),
            pl.BlockSpec((1, ROW_BLK, FH), lambda i: (0, i, 0)),
            pl.BlockSpec((F, F), lambda i: (0, 0)),
        ],
        out_specs=pl.BlockSpec((2, ROW_BLK, FH), lambda i: (0, i, 0)),
        out_shape=jax.ShapeDtypeStruct((2, N_ACC, FH), jnp.float32),
    )(h_pad, deg, W)


def _mm2_body(a_ref, deg_ref, b_ref, w_ref, out_ref):
    i = pl.program_id(0)
    x = jnp.concatenate([a_ref[0], a_ref[1]], axis=1)
    norm_dst = _norm(deg_ref[1])
    norm_src = _norm(deg_ref[0])
    x1 = jax.nn.relu(x * norm_dst + b_ref[...])
    rows = i * ROW_BLK + lax.broadcasted_iota(jnp.int32, (ROW_BLK, 1), 0)
    x1 = jnp.where(rows < N, x1, 0.0)
    y = jnp.dot(x1 * norm_src, w_ref[...], preferred_element_type=jnp.float32)
    out_ref[0] = y[:, :FH]
    out_ref[1] = y[:, FH:]


def _mm2(a1, deg, b1, W):
    return pl.pallas_call(
        _mm2_body,
        grid=(N_BLKS,),
        in_specs=[
            pl.BlockSpec((2, ROW_BLK, FH), lambda i: (0, i, 0)),
            pl.BlockSpec((2, ROW_BLK, FH), lambda i: (0, i, 0)),
            pl.BlockSpec((1, F), lambda i: (0, 0)),
            pl.BlockSpec((F, F), lambda i: (0, 0)),
        ],
        out_specs=pl.BlockSpec((2, ROW_BLK, FH), lambda i: (0, i, 0)),
        out_shape=jax.ShapeDtypeStruct((2, N_ACC, FH), jnp.float32),
    )(a1, deg, b1, W)


def _final_body(a_ref, deg_ref, b_ref, out_ref):
    x = jnp.concatenate([a_ref[0], a_ref[1]], axis=1)
    out_ref[...] = x * _norm(deg_ref[1]) + b_ref[...]


def _final(a2, deg, b2):
    return pl.pallas_call(
        _final_body,
        grid=(N_BLKS,),
        in_specs=[
            pl.BlockSpec((2, ROW_BLK, FH), lambda i: (0, i, 0)),
            pl.BlockSpec((2, ROW_BLK, FH), lambda i: (0, i, 0)),
            pl.BlockSpec((1, F), lambda i: (0, 0)),
        ],
        out_specs=pl.BlockSpec((ROW_BLK, F), lambda i: (i, 0)),
        out_shape=jax.ShapeDtypeStruct((N, F), jnp.float32),
    )(a2, deg, b2)


def kernel(h, edge_index, W1, b1, W2, b2):
    src = edge_index[0]
    dst = edge_index[1]
    pad = jnp.full((E_PAD - E,), N, dtype=jnp.int32)
    src_r = jnp.concatenate([src, pad])
    dst_r = jnp.concatenate([dst, pad])
    edges = jnp.stack([src_r, dst_r])

    ones_vals = jnp.ones((SLAB, FH), jnp.float32)
    zeros_slab = jnp.zeros((ROWS_PER_TILE, FH), jnp.float32)
    h_pad = jnp.concatenate([h, jnp.zeros((N_ACC - N, F), h.dtype)], axis=0)

    deg = _deg_kernel(edges, ones_vals, zeros_slab)
    y1 = _mm1(h_pad, deg, W1)
    a1 = _agg_kernel(y1, edges, zeros_slab)
    y2 = _mm2(a1, deg, b1.reshape(1, F), W2)
    a2 = _agg_kernel(y2, edges, zeros_slab)
    return _final(a2, deg, b2.reshape(1, F))


# speedup vs baseline: 1.0577x; 1.0291x over previous
import functools

import jax
import jax.numpy as jnp
from jax import lax
from jax.experimental import pallas as pl
from jax.experimental.pallas import tpu as pltpu
from jax.experimental.pallas import tpu_sc as plsc

N = 10000
E = 160000
F = 256
FH = 128

N_ACC = 10112
ROW_BLK = 128
N_BLKS = N_ACC // ROW_BLK
TILES = 16
ROWS_PER_TILE = N_ACC // TILES

E_PAD = 163840
SLAB = 128
N_SLABS = E_PAD // SLAB
SLABS_PER_TILE = N_SLABS // TILES
IDXC = 16

GS = 64
GSPT = E_PAD // (TILES * GS)
GIDXC = 40

_mesh = plsc.VectorSubcoreMesh(core_axis_name="c", subcore_axis_name="s")


@functools.partial(
    pl.kernel,
    out_type=jax.ShapeDtypeStruct((2, N_ACC, FH), jnp.float32),
    mesh=_mesh,
    scratch_types=[
        pltpu.VMEM((SLABS_PER_TILE * SLAB,), jnp.int32),
        pltpu.VMEM((SLAB, FH), jnp.float32),
        pltpu.VMEM_SHARED((N_ACC, FH), jnp.float32),
    ],
)
def _deg_kernel(edges_hbm, ones_hbm, zeros_hbm, out_hbm, idx_v, ones_v, acc):
    c = lax.axis_index("c")
    s = lax.axis_index("s")
    rbase = s * ROWS_PER_TILE
    ebase = s * SLABS_PER_TILE * SLAB
    pltpu.sync_copy(zeros_hbm, acc.at[pl.ds(rbase, ROWS_PER_TILE)])
    pltpu.sync_copy(ones_hbm, ones_v)

    @pl.when(c == 0)
    def _():
        pltpu.sync_copy(edges_hbm.at[0].at[pl.ds(ebase, SLABS_PER_TILE * SLAB)],
                        idx_v)

    @pl.when(c == 1)
    def _():
        pltpu.sync_copy(edges_hbm.at[1].at[pl.ds(ebase, SLABS_PER_TILE * SLAB)],
                        idx_v)

    plsc.subcore_barrier()

    def body(j, _):
        pltpu.sync_copy(ones_v, acc.at[idx_v.at[pl.ds(j * SLAB, SLAB)]], add=True)
        return ()

    lax.fori_loop(0, SLABS_PER_TILE, body, ())
    plsc.subcore_barrier()

    @pl.when(c == 0)
    def _():
        pltpu.sync_copy(acc.at[pl.ds(rbase, ROWS_PER_TILE)],
                        out_hbm.at[0].at[pl.ds(rbase, ROWS_PER_TILE)])

    @pl.when(c == 1)
    def _():
        pltpu.sync_copy(acc.at[pl.ds(rbase, ROWS_PER_TILE)],
                        out_hbm.at[1].at[pl.ds(rbase, ROWS_PER_TILE)])


@functools.partial(
    pl.kernel,
    out_type=jax.ShapeDtypeStruct((2, N_ACC, FH), jnp.float32),
    mesh=_mesh,
    scratch_types=[
        pltpu.VMEM((GIDXC * GS,), jnp.int32),
        pltpu.VMEM((GIDXC * GS,), jnp.int32),
        pltpu.VMEM((GS, FH), jnp.float32),
        pltpu.VMEM((GS, FH), jnp.float32),
        pltpu.VMEM((GS, FH), jnp.float32),
        pltpu.VMEM((GS, FH), jnp.float32),
        pltpu.VMEM_SHARED((N_ACC, FH), jnp.float32),
        pltpu.SemaphoreType.DMA,
        pltpu.SemaphoreType.DMA,
    ],
)
def _agg_kernel(y_hbm, edges_hbm, zeros_hbm, out_hbm,
                src_c, dst_c, b0, b1, b2, b3, acc, gsem, ssem):
    bufs = (b0, b1, b2, b3)
    c = lax.axis_index("c")
    s = lax.axis_index("s")
    rbase = s * ROWS_PER_TILE
    pltpu.sync_copy(zeros_hbm, acc.at[pl.ds(rbase, ROWS_PER_TILE)])
    plsc.subcore_barrier()

    def _gather(j, buf):
        pltpu.async_copy(
            y_hbm.at[c].at[src_c.at[pl.ds(j * GS, GS)]], buf, gsem)

    def _gwait(j, buf):
        pltpu.make_async_copy(
            y_hbm.at[c].at[src_c.at[pl.ds(j * GS, GS)]], buf, gsem).wait()

    def _sfire(j, buf):
        pltpu.async_copy(buf, acc.at[dst_c.at[pl.ds(j * GS, GS)]], ssem,
                         add=True)

    def _swait(j, buf):
        pltpu.make_async_copy(buf, acc.at[dst_c.at[pl.ds(j * GS, GS)]],
                              ssem).wait()

    NBUF = len(bufs)

    def chunk_body(k, _):
        ebase = (s * GSPT + k * GIDXC) * GS
        pltpu.sync_copy(edges_hbm.at[0].at[pl.ds(ebase, GIDXC * GS)], src_c)
        pltpu.sync_copy(edges_hbm.at[1].at[pl.ds(ebase, GIDXC * GS)], dst_c)
        for b in range(NBUF):
            _gather(b, bufs[b])

        def body(jj, _):
            for b in range(NBUF):
                j = NBUF * jj + b
                _gwait(j, bufs[b])
                _sfire(j, bufs[b])
                _swait(j, bufs[b])

                @pl.when(jj < GIDXC // NBUF - 1)
                def _():
                    _gather(j + NBUF, bufs[b])

            return ()

        lax.fori_loop(0, GIDXC // NBUF, body, ())
        return ()

    lax.fori_loop(0, GSPT // GIDXC, chunk_body, ())
    plsc.subcore_barrier()
    pltpu.sync_copy(acc.at[pl.ds(rbase, ROWS_PER_TILE)],
                    out_hbm.at[c].at[pl.ds(rbase, ROWS_PER_TILE)])


def _norm(deg_blk):
    return lax.rsqrt(jnp.clip(deg_blk[:, :1], 1.0, None))


def _mm1_body(h_ref, deg_ref, w_ref, out_ref):
    x = h_ref[...] * _norm(deg_ref[0])
    y = jnp.dot(x, w_ref[...], preferred_element_type=jnp.float32)
    out_ref[0] = y[:, :FH]
    out_ref[1] = y[:, FH:]


def _mm1(h_pad, deg, W):
    return pl.pallas_call(
        _mm1_body,
        grid=(N_BLKS,),
        in_specs=[
            pl.BlockSpec((ROW_BLK, F), lambda i: (i, 0)),
            pl.BlockSpec((1, ROW_BLK, FH), lambda i: (0, i, 0)),
            pl.BlockSpec((F, F), lambda i: (0, 0)),
        ],
        out_specs=pl.BlockSpec((2, ROW_BLK, FH), lambda i: (0, i, 0)),
        out_shape=jax.ShapeDtypeStruct((2, N_ACC, FH), jnp.float32),
    )(h_pad, deg, W)


def _mm2_body(a_ref, deg_ref, b_ref, w_ref, out_ref):
    i = pl.program_id(0)
    x = jnp.concatenate([a_ref[0], a_ref[1]], axis=1)
    norm_dst = _norm(deg_ref[1])
    norm_src = _norm(deg_ref[0])
    x1 = jax.nn.relu(x * norm_dst + b_ref[...])
    rows = i * ROW_BLK + lax.broadcasted_iota(jnp.int32, (ROW_BLK, 1), 0)
    x1 = jnp.where(rows < N, x1, 0.0)
    y = jnp.dot(x1 * norm_src, w_ref[...], preferred_element_type=jnp.float32)
    out_ref[0] = y[:, :FH]
    out_ref[1] = y[:, FH:]


def _mm2(a1, deg, b1, W):
    return pl.pallas_call(
        _mm2_body,
        grid=(N_BLKS,),
        in_specs=[
            pl.BlockSpec((2, ROW_BLK, FH), lambda i: (0, i, 0)),
            pl.BlockSpec((2, ROW_BLK, FH), lambda i: (0, i, 0)),
            pl.BlockSpec((1, F), lambda i: (0, 0)),
            pl.BlockSpec((F, F), lambda i: (0, 0)),
        ],
        out_specs=pl.BlockSpec((2, ROW_BLK, FH), lambda i: (0, i, 0)),
        out_shape=jax.ShapeDtypeStruct((2, N_ACC, FH), jnp.float32),
    )(a1, deg, b1, W)


def _final_body(a_ref, deg_ref, b_ref, out_ref):
    x = jnp.concatenate([a_ref[0], a_ref[1]], axis=1)
    out_ref[...] = x * _norm(deg_ref[1]) + b_ref[...]


def _final(a2, deg, b2):
    return pl.pallas_call(
        _final_body,
        grid=(N_BLKS,),
        in_specs=[
            pl.BlockSpec((2, ROW_BLK, FH), lambda i: (0, i, 0)),
            pl.BlockSpec((2, ROW_BLK, FH), lambda i: (0, i, 0)),
            pl.BlockSpec((1, F), lambda i: (0, 0)),
        ],
        out_specs=pl.BlockSpec((ROW_BLK, F), lambda i: (i, 0)),
        out_shape=jax.ShapeDtypeStruct((N, F), jnp.float32),
    )(a2, deg, b2)


def kernel(h, edge_index, W1, b1, W2, b2):
    src = edge_index[0]
    dst = edge_index[1]
    pad = jnp.full((E_PAD - E,), N, dtype=jnp.int32)
    src_r = jnp.concatenate([src, pad])
    dst_r = jnp.concatenate([dst, pad])
    edges = jnp.stack([src_r, dst_r])

    ones_vals = jnp.ones((SLAB, FH), jnp.float32)
    zeros_slab = jnp.zeros((ROWS_PER_TILE, FH), jnp.float32)
    h_pad = jnp.concatenate([h, jnp.zeros((N_ACC - N, F), h.dtype)], axis=0)

    deg = _deg_kernel(edges, ones_vals, zeros_slab)
    y1 = _mm1(h_pad, deg, W1)
    a1 = _agg_kernel(y1, edges, zeros_slab)
    y2 = _mm2(a1, deg, b1.reshape(1, F), W2)
    a2 = _agg_kernel(y2, edges, zeros_slab)
    return _final(a2, deg, b2.reshape(1, F))


# speedup vs baseline: 1.0635x; 1.0054x over previous
import functools

import jax
import jax.numpy as jnp
from jax import lax
from jax.experimental import pallas as pl
from jax.experimental.pallas import tpu as pltpu
from jax.experimental.pallas import tpu_sc as plsc

N = 10000
E = 160000
F = 256
FH = 128

N_ACC = 10112
ROW_BLK = 128
N_BLKS = N_ACC // ROW_BLK
TILES = 16
ROWS_PER_TILE = N_ACC // TILES

E_PAD = 163840
SLAB = 128
N_SLABS = E_PAD // SLAB
SLABS_PER_TILE = N_SLABS // TILES
IDXC = 16

GS = 64
GSPT = E_PAD // (TILES * GS)
GIDXC = 80

_mesh = plsc.VectorSubcoreMesh(core_axis_name="c", subcore_axis_name="s")


@functools.partial(
    pl.kernel,
    out_type=jax.ShapeDtypeStruct((2, N_ACC, FH), jnp.float32),
    mesh=_mesh,
    scratch_types=[
        pltpu.VMEM((SLABS_PER_TILE * SLAB,), jnp.int32),
        pltpu.VMEM((SLAB, FH), jnp.float32),
        pltpu.VMEM_SHARED((N_ACC, FH), jnp.float32),
    ],
)
def _deg_kernel(edges_hbm, ones_hbm, zeros_hbm, out_hbm, idx_v, ones_v, acc):
    c = lax.axis_index("c")
    s = lax.axis_index("s")
    rbase = s * ROWS_PER_TILE
    ebase = s * SLABS_PER_TILE * SLAB
    pltpu.sync_copy(zeros_hbm, acc.at[pl.ds(rbase, ROWS_PER_TILE)])
    pltpu.sync_copy(ones_hbm, ones_v)

    @pl.when(c == 0)
    def _():
        pltpu.sync_copy(edges_hbm.at[0].at[pl.ds(ebase, SLABS_PER_TILE * SLAB)],
                        idx_v)

    @pl.when(c == 1)
    def _():
        pltpu.sync_copy(edges_hbm.at[1].at[pl.ds(ebase, SLABS_PER_TILE * SLAB)],
                        idx_v)

    plsc.subcore_barrier()

    def body(j, _):
        pltpu.sync_copy(ones_v, acc.at[idx_v.at[pl.ds(j * SLAB, SLAB)]], add=True)
        return ()

    lax.fori_loop(0, SLABS_PER_TILE, body, ())
    plsc.subcore_barrier()

    @pl.when(c == 0)
    def _():
        pltpu.sync_copy(acc.at[pl.ds(rbase, ROWS_PER_TILE)],
                        out_hbm.at[0].at[pl.ds(rbase, ROWS_PER_TILE)])

    @pl.when(c == 1)
    def _():
        pltpu.sync_copy(acc.at[pl.ds(rbase, ROWS_PER_TILE)],
                        out_hbm.at[1].at[pl.ds(rbase, ROWS_PER_TILE)])


@functools.partial(
    pl.kernel,
    out_type=jax.ShapeDtypeStruct((2, N_ACC, FH), jnp.float32),
    mesh=_mesh,
    scratch_types=[
        pltpu.VMEM((GIDXC * GS,), jnp.int32),
        pltpu.VMEM((GIDXC * GS,), jnp.int32),
        pltpu.VMEM((GS, FH), jnp.float32),
        pltpu.VMEM((GS, FH), jnp.float32),
        pltpu.VMEM((GS, FH), jnp.float32),
        pltpu.VMEM((GS, FH), jnp.float32),
        pltpu.VMEM_SHARED((N_ACC, FH), jnp.float32),
        pltpu.SemaphoreType.DMA,
        pltpu.SemaphoreType.DMA,
    ],
)
def _agg_kernel(y_hbm, edges_hbm, zeros_hbm, out_hbm,
                src_c, dst_c, b0, b1, b2, b3, acc, gsem, ssem):
    bufs = (b0, b1, b2, b3)
    c = lax.axis_index("c")
    s = lax.axis_index("s")
    rbase = s * ROWS_PER_TILE
    pltpu.sync_copy(zeros_hbm, acc.at[pl.ds(rbase, ROWS_PER_TILE)])
    plsc.subcore_barrier()

    def _gather(j, buf):
        pltpu.async_copy(
            y_hbm.at[c].at[src_c.at[pl.ds(j * GS, GS)]], buf, gsem)

    def _gwait(j, buf):
        pltpu.make_async_copy(
            y_hbm.at[c].at[src_c.at[pl.ds(j * GS, GS)]], buf, gsem).wait()

    def _sfire(j, buf):
        pltpu.async_copy(buf, acc.at[dst_c.at[pl.ds(j * GS, GS)]], ssem,
                         add=True)

    def _swait(j, buf):
        pltpu.make_async_copy(buf, acc.at[dst_c.at[pl.ds(j * GS, GS)]],
                              ssem).wait()

    NBUF = len(bufs)

    def chunk_body(k, _):
        ebase = (s * GSPT + k * GIDXC) * GS
        pltpu.sync_copy(edges_hbm.at[0].at[pl.ds(ebase, GIDXC * GS)], src_c)
        pltpu.sync_copy(edges_hbm.at[1].at[pl.ds(ebase, GIDXC * GS)], dst_c)
        for b in range(NBUF):
            _gather(b, bufs[b])

        def body(jj, _):
            for b in range(NBUF):
                j = NBUF * jj + b
                _gwait(j, bufs[b])
                _sfire(j, bufs[b])
                _swait(j, bufs[b])

                @pl.when(jj < GIDXC // NBUF - 1)
                def _():
                    _gather(j + NBUF, bufs[b])

            return ()

        lax.fori_loop(0, GIDXC // NBUF, body, ())
        return ()

    lax.fori_loop(0, GSPT // GIDXC, chunk_body, ())
    plsc.subcore_barrier()
    pltpu.sync_copy(acc.at[pl.ds(rbase, ROWS_PER_TILE)],
                    out_hbm.at[c].at[pl.ds(rbase, ROWS_PER_TILE)])


def _norm(deg_blk):
    return lax.rsqrt(jnp.clip(deg_blk[:, :1], 1.0, None))


def _mm1_body(h_ref, deg_ref, w_ref, out_ref):
    x = h_ref[...] * _norm(deg_ref[0])
    y = jnp.dot(x, w_ref[...], preferred_element_type=jnp.float32)
    out_ref[0] = y[:, :FH]
    out_ref[1] = y[:, FH:]


def _mm1(h_pad, deg, W):
    return pl.pallas_call(
        _mm1_body,
        grid=(N_BLKS,),
        in_specs=[
            pl.BlockSpec((ROW_BLK, F), lambda i: (i, 0)),
            pl.BlockSpec((1, ROW_BLK, FH), lambda i: (0, i, 0)),
            pl.BlockSpec((F, F), lambda i: (0, 0)),
        ],
        out_specs=pl.BlockSpec((2, ROW_BLK, FH), lambda i: (0, i, 0)),
        out_shape=jax.ShapeDtypeStruct((2, N_ACC, FH), jnp.float32),
    )(h_pad, deg, W)


def _mm2_body(a_ref, deg_ref, b_ref, w_ref, out_ref):
    i = pl.program_id(0)
    x = jnp.concatenate([a_ref[0], a_ref[1]], axis=1)
    norm_dst = _norm(deg_ref[1])
    norm_src = _norm(deg_ref[0])
    x1 = jax.nn.relu(x * norm_dst + b_ref[...])
    rows = i * ROW_BLK + lax.broadcasted_iota(jnp.int32, (ROW_BLK, 1), 0)
    x1 = jnp.where(rows < N, x1, 0.0)
    y = jnp.dot(x1 * norm_src, w_ref[...], preferred_element_type=jnp.float32)
    out_ref[0] = y[:, :FH]
    out_ref[1] = y[:, FH:]


def _mm2(a1, deg, b1, W):
    return pl.pallas_call(
        _mm2_body,
        grid=(N_BLKS,),
        in_specs=[
            pl.BlockSpec((2, ROW_BLK, FH), lambda i: (0, i, 0)),
            pl.BlockSpec((2, ROW_BLK, FH), lambda i: (0, i, 0)),
            pl.BlockSpec((1, F), lambda i: (0, 0)),
            pl.BlockSpec((F, F), lambda i: (0, 0)),
        ],
        out_specs=pl.BlockSpec((2, ROW_BLK, FH), lambda i: (0, i, 0)),
        out_shape=jax.ShapeDtypeStruct((2, N_ACC, FH), jnp.float32),
    )(a1, deg, b1, W)


def _final_body(a_ref, deg_ref, b_ref, out_ref):
    x = jnp.concatenate([a_ref[0], a_ref[1]], axis=1)
    out_ref[...] = x * _norm(deg_ref[1]) + b_ref[...]


def _final(a2, deg, b2):
    return pl.pallas_call(
        _final_body,
        grid=(N_BLKS,),
        in_specs=[
            pl.BlockSpec((2, ROW_BLK, FH), lambda i: (0, i, 0)),
            pl.BlockSpec((2, ROW_BLK, FH), lambda i: (0, i, 0)),
            pl.BlockSpec((1, F), lambda i: (0, 0)),
        ],
        out_specs=pl.BlockSpec((ROW_BLK, F), lambda i: (i, 0)),
        out_shape=jax.ShapeDtypeStruct((N, F), jnp.float32),
    )(a2, deg, b2)


def kernel(h, edge_index, W1, b1, W2, b2):
    src = edge_index[0]
    dst = edge_index[1]
    pad = jnp.full((E_PAD - E,), N, dtype=jnp.int32)
    src_r = jnp.concatenate([src, pad])
    dst_r = jnp.concatenate([dst, pad])
    edges = jnp.stack([src_r, dst_r])

    ones_vals = jnp.ones((SLAB, FH), jnp.float32)
    zeros_slab = jnp.zeros((ROWS_PER_TILE, FH), jnp.float32)
    h_pad = jnp.concatenate([h, jnp.zeros((N_ACC - N, F), h.dtype)], axis=0)

    deg = _deg_kernel(edges, ones_vals, zeros_slab)
    y1 = _mm1(h_pad, deg, W1)
    a1 = _agg_kernel(y1, edges, zeros_slab)
    y2 = _mm2(a1, deg, b1.reshape(1, F), W2)
    a2 = _agg_kernel(y2, edges, zeros_slab)
    return _final(a2, deg, b2.reshape(1, F))
